# baseline (speedup 1.0000x reference)
import functools

import jax
import jax.numpy as jnp
from jax import lax
from jax.experimental import pallas as pl
from jax.experimental.pallas import tpu as pltpu
from jax.experimental.pallas import tpu_sc as plsc

_DIM = 64
_NC, _NS, _L = 2, 16, 16
_NW = _NC * _NS
_C = 64
_NSL = 4
_ZB = 2048
_LR = 4
_LT = 2


def _body(idx_hbm, map_hbm, table_hbm, out_hbm, idx_v, smap, row_sl, rbuf,
          msem, gsem, wsem, *, nch, nrows, vocab):
    sid = lax.axis_index("s")
    wid = sid * _NC + lax.axis_index("c")

    @pl.when(sid == 0)
    def _stage_map():
        pltpu.sync_copy(map_hbm, smap)

    pltpu.sync_copy(idx_hbm.at[wid], idx_v)
    plsc.subcore_barrier()

    lanes = lax.iota(jnp.int32, _L)

    def remap_fire(j, slot):
        for k in range(_C // _L):
            v = idx_v[j, pl.ds(k * _L, _L)]
            idx_v[j, pl.ds(k * _L, _L)] = jnp.minimum(
                jnp.maximum(v, 0), vocab)
        pltpu.make_async_copy(
            smap.at[idx_v.at[j]], row_sl.at[slot], msem.at[slot]).start()

    def remap_wait(j, slot):
        pltpu.make_async_copy(
            smap.at[idx_v.at[j]], row_sl.at[slot], msem.at[slot]).wait()
        for k in range(_C // _L):
            r = row_sl[slot, pl.ds(k * _L, _L)]
            spread = jnp.bitwise_and(
                j * _C + k * _L + lanes + wid * 997, _ZB - 1)
            row_sl[slot, pl.ds(k * _L, _L)] = jnp.where(
                r == nrows, nrows + 1 + spread, r)

    def g_copy(slot):
        return pltpu.make_async_copy(
            table_hbm.at[row_sl.at[slot]], rbuf.at[slot], gsem.at[slot])

    def w_copy(j, slot):
        return pltpu.make_async_copy(
            rbuf.at[slot, :, pl.ds(0, _DIM)],
            out_hbm.at[pl.ds((wid * nch + j) * _C, _C)], wsem.at[slot])

    for c in range(_LR):
        remap_fire(c, c % _NSL)
    for c in range(_LT):
        remap_wait(c, c % _NSL)
        g_copy(c % _NSL).start()

    def main(g, carry):
        j0 = g * _NSL
        for b in range(_NSL):
            j = j0 + b
            g_copy(b).wait()
            w_copy(j, b).start()

            jt = j + _LT

            @pl.when(jt < nch)
            def _advance():
                remap_wait(jt, (b + _LT) % _NSL)

                @pl.when(j >= _LT)
                def _retire():
                    w_copy(j - _LT, (b - _LT) % _NSL).wait()

                g_copy((b + _LT) % _NSL).start()

            jr = j + _LR

            @pl.when(jr < nch)
            def _refire():
                remap_fire(jr, (b + _LR) % _NSL)

        return carry

    lax.fori_loop(0, nch // _NSL, main, 0)

    for i in range(2 * _LT):
        j = nch - 2 * _LT + i
        w_copy(j, j % _NSL).wait()


@functools.partial(jax.jit, static_argnames=("nch", "nrows", "vocab"))
def _lookup(idx, map1, table, *, nch, nrows, vocab):
    n = _NW * nch * _C
    body = functools.partial(_body, nch=nch, nrows=nrows, vocab=vocab)
    return pl.kernel(
        body,
        out_type=jax.ShapeDtypeStruct((n, _DIM), jnp.float32),
        mesh=plsc.VectorSubcoreMesh(
            core_axis_name="c", subcore_axis_name="s",
            num_cores=_NC, num_subcores=_NS),
        scratch_types=[
            pltpu.VMEM((nch, _C), jnp.int32),
            pltpu.VMEM_SHARED(map1.shape, jnp.int32),
            pltpu.VMEM((_NSL, _C), jnp.int32),
            pltpu.VMEM((_NSL, _C, 2 * _DIM), jnp.float32),
            pltpu.SemaphoreType.DMA((_NSL,)),
            pltpu.SemaphoreType.DMA((_NSL,)),
            pltpu.SemaphoreType.DMA((_NSL,)),
        ],
        compiler_params=pltpu.CompilerParams(
            use_tc_tiling_on_sc=False, needs_layout_passes=False),
    )(idx, map1, table)


def kernel(index, table, index_map):
    b, f = index.shape
    n = b * f
    nch = n // (_NW * _C)
    assert nch * _NW * _C == n and nch % _NSL == 0
    idx = index.reshape(_NW, nch, _C)
    pad = (-index_map.shape[0]) % 16
    map1 = jnp.concatenate([index_map, jnp.zeros((pad,), jnp.int32)])
    tab = jnp.pad(jnp.concatenate(
        [table, jnp.zeros((_ZB, table.shape[1]), table.dtype)]),
        ((0, 0), (0, table.shape[1])))
    out = _lookup(idx, map1, tab, nch=nch, nrows=table.shape[0] - 1,
                  vocab=index_map.shape[0] - 1)
    return out.reshape(b, f, _DIM)


# speedup vs baseline: 1.1006x; 1.1006x over previous
import functools

import jax
import jax.numpy as jnp
from jax import lax
from jax.experimental import pallas as pl
from jax.experimental.pallas import tpu as pltpu
from jax.experimental.pallas import tpu_sc as plsc

_DIM = 64
_NC, _NS, _L = 2, 16, 16
_NW = _NC * _NS
_C = 64
_NSL = 8
_ZB = 2048
_LR = 8
_LT = 4


def _body(idx_hbm, map_hbm, table_hbm, out_hbm, idx_v, smap, row_sl, rbuf,
          msem, gsem, wsem, *, nch, nrows, vocab):
    sid = lax.axis_index("s")
    wid = sid * _NC + lax.axis_index("c")

    @pl.when(sid == 0)
    def _stage_map():
        pltpu.sync_copy(map_hbm, smap)

    pltpu.sync_copy(idx_hbm.at[wid], idx_v)
    plsc.subcore_barrier()

    lanes = lax.iota(jnp.int32, _L)

    def remap_fire(j, slot):
        for k in range(_C // _L):
            v = idx_v[j, pl.ds(k * _L, _L)]
            idx_v[j, pl.ds(k * _L, _L)] = jnp.minimum(
                jnp.maximum(v, 0), vocab)
        pltpu.make_async_copy(
            smap.at[idx_v.at[j]], row_sl.at[slot], msem.at[slot]).start()

    def remap_wait(j, slot):
        pltpu.make_async_copy(
            smap.at[idx_v.at[j]], row_sl.at[slot], msem.at[slot]).wait()
        for k in range(_C // _L):
            r = row_sl[slot, pl.ds(k * _L, _L)]
            spread = jnp.bitwise_and(
                j * _C + k * _L + lanes + wid * 997, _ZB - 1)
            row_sl[slot, pl.ds(k * _L, _L)] = jnp.where(
                r == nrows, nrows + 1 + spread, r)

    def g_copy(slot):
        return pltpu.make_async_copy(
            table_hbm.at[row_sl.at[slot]], rbuf.at[slot], gsem.at[slot])

    def w_copy(j, slot):
        return pltpu.make_async_copy(
            rbuf.at[slot], out_hbm.at[pl.ds((wid * nch + j) * _C, _C)],
            wsem.at[slot])

    for c in range(_LR):
        remap_fire(c, c % _NSL)
    for c in range(_LT):
        remap_wait(c, c % _NSL)
        g_copy(c % _NSL).start()

    def main(g, carry):
        j0 = g * _NSL
        for b in range(_NSL):
            j = j0 + b
            g_copy(b).wait()
            w_copy(j, b).start()

            jt = j + _LT

            @pl.when(jt < nch)
            def _advance():
                remap_wait(jt, (b + _LT) % _NSL)

                @pl.when(j >= _LT)
                def _retire():
                    w_copy(j - _LT, (b - _LT) % _NSL).wait()

                g_copy((b + _LT) % _NSL).start()

            jr = j + _LR

            @pl.when(jr < nch)
            def _refire():
                remap_fire(jr, (b + _LR) % _NSL)

        return carry

    lax.fori_loop(0, nch // _NSL, main, 0)

    for i in range(2 * _LT):
        j = nch - 2 * _LT + i
        w_copy(j, j % _NSL).wait()


@functools.partial(jax.jit, static_argnames=("nch", "nrows", "vocab"))
def _lookup(idx, map1, table, *, nch, nrows, vocab):
    n = _NW * nch * _C
    body = functools.partial(_body, nch=nch, nrows=nrows, vocab=vocab)
    return pl.kernel(
        body,
        out_type=jax.ShapeDtypeStruct((n, _DIM), jnp.float32),
        mesh=plsc.VectorSubcoreMesh(
            core_axis_name="c", subcore_axis_name="s",
            num_cores=_NC, num_subcores=_NS),
        scratch_types=[
            pltpu.VMEM((nch, _C), jnp.int32),
            pltpu.VMEM_SHARED(map1.shape, jnp.int32),
            pltpu.VMEM((_NSL, _C), jnp.int32),
            pltpu.VMEM((_NSL, _C, _DIM), jnp.float32),
            pltpu.SemaphoreType.DMA((_NSL,)),
            pltpu.SemaphoreType.DMA((_NSL,)),
            pltpu.SemaphoreType.DMA((_NSL,)),
        ],
        compiler_params=pltpu.CompilerParams(
            use_tc_tiling_on_sc=False, needs_layout_passes=False),
    )(idx, map1, table)


def kernel(index, table, index_map):
    b, f = index.shape
    n = b * f
    nch = n // (_NW * _C)
    assert nch * _NW * _C == n and nch % _NSL == 0
    idx = index.reshape(_NW, nch, _C)
    pad = (-index_map.shape[0]) % 16
    map1 = jnp.concatenate([index_map, jnp.zeros((pad,), jnp.int32)])
    tab = jnp.concatenate([table, jnp.zeros((_ZB, table.shape[1]),
                                            table.dtype)])
    out = _lookup(idx, map1, tab, nch=nch, nrows=table.shape[0] - 1,
                  vocab=index_map.shape[0] - 1)
    return out.reshape(b, f, _DIM)
